# baseline (speedup 1.0000x reference)
import functools

import jax
import jax.numpy as jnp
from jax import lax
from jax.experimental import pallas as pl
from jax.experimental.pallas import tpu as pltpu
from jax.experimental.pallas import tpu_sc as plsc

N_NODES = 10000
NFEAT = 256
NHID = 256
NCLASS = 64

NC = 2
NS = 16
NW = NC * NS
LANES = 16
CHUNK = 32

ROW_BLK = 1000


KCH = 4


def _make_spmm(d: int, n0: int, n1: int):
    nmax = max(n0, n1)
    rows_per_sub = 624
    tail_start = NS * rows_per_sub
    tail_rows = N_NODES - tail_start
    mesh = plsc.VectorSubcoreMesh(
        core_axis_name="c", subcore_axis_name="s", num_cores=NC, num_subcores=NS
    )

    @functools.partial(
        pl.kernel,
        out_type=jax.ShapeDtypeStruct((NC, N_NODES, d), jnp.float32),
        mesh=mesh,
        scratch_types=[
            pltpu.VMEM((KCH, CHUNK), jnp.int32),
            [pltpu.VMEM((CHUNK,), jnp.int32)] * KCH,
            pltpu.VMEM((KCH * CHUNK,), jnp.float32),
            [pltpu.VMEM((CHUNK, d), jnp.float32)] * KCH,
            pltpu.VMEM_SHARED((N_NODES, d), jnp.float32),
            [pltpu.SemaphoreType.DMA] * KCH,
            pltpu.SemaphoreType.DMA,
            [pltpu.SemaphoreType.DMA] * KCH,
            pltpu.SemaphoreType.DMA,
            pltpu.SemaphoreType.DMA,
        ],
    )
    def spmm(h_hbm, src_hbm, dst_hbm, w_hbm, out_hbm, src_blk,
             dst_bufs, w_all, bufs, acc, gsems, wsem, ssems, dsem, srcsem):
        cid = lax.axis_index("c")
        sid = lax.axis_index("s")
        wid = sid * NC + cid

        def _zero_row(i, carry):
            for j in range(d // LANES):
                bufs[0][i, pl.ds(j * LANES, LANES)] = jnp.zeros(
                    (LANES,), jnp.float32)
            return carry
        lax.fori_loop(0, CHUNK, _zero_row, 0)
        r0 = sid * rows_per_sub
        for t in range(19):
            pltpu.sync_copy(bufs[0],
                            acc.at[pl.ds(r0 + t * CHUNK, CHUNK)])
        pltpu.sync_copy(bufs[0].at[pl.ds(0, 16)],
                        acc.at[pl.ds(r0 + 19 * CHUNK, 16)])

        @pl.when(sid == NS - 1)
        def _zero_tail():
            pltpu.sync_copy(bufs[0].at[pl.ds(0, tail_rows)],
                            acc.at[pl.ds(tail_start, tail_rows)])

        plsc.subcore_barrier()

        def _scale(i_buf):
            for g16 in range(CHUNK // LANES):
                wreg = w_all[pl.ds(i_buf * CHUNK + g16 * LANES, LANES)]

                def _row(r, c2, _wreg=wreg, _g16=g16):
                    idx = jnp.broadcast_to(r, (LANES, 1)).astype(jnp.int32)
                    wb = lax.gather(
                        _wreg, idx,
                        lax.GatherDimensionNumbers(
                            offset_dims=(), collapsed_slice_dims=(0,),
                            start_index_map=(0,)),
                        slice_sizes=(1,),
                        mode=lax.GatherScatterMode.PROMISE_IN_BOUNDS)
                    row = _g16 * LANES + r
                    for j in range(d // LANES):
                        sl = pl.ds(j * LANES, LANES)
                        bufs[i_buf][row, sl] = bufs[i_buf][row, sl] * wb
                    return c2
                lax.fori_loop(0, LANES, _row, 0)

        def _body(t, carry):
            base = t * KCH
            sds = [pltpu.async_copy(src_hbm.at[wid, base + i],
                                    src_blk.at[i], srcsem)
                   for i in range(KCH)]
            wd = pltpu.async_copy(
                w_hbm.at[wid, pl.ds(base * CHUNK, KCH * CHUNK)], w_all, wsem)
            dds = [pltpu.async_copy(dst_hbm.at[wid, base + i],
                                    dst_bufs[i], dsem)
                   for i in range(KCH)]
            gds = []
            for i in range(KCH):
                sds[i].wait()
                gds.append(pltpu.async_copy(h_hbm.at[src_blk.at[i]], bufs[i],
                                            gsems[i]))
            wd.wait()
            for dd in dds:
                dd.wait()
            scats = []
            for i in range(KCH):
                gds[i].wait()
                _scale(i)
                if scats:
                    scats[-1].wait()
                scats.append(pltpu.async_copy(bufs[i], acc.at[dst_bufs[i]],
                                              ssems[i], add=True))
            scats[-1].wait()
            return carry
        n_my = jnp.where(cid == 0, n0 // KCH, n1 // KCH)
        lax.fori_loop(0, n_my, _body, 0)

        plsc.subcore_barrier()
        pltpu.sync_copy(acc.at[pl.ds(r0, rows_per_sub)],
                        out_hbm.at[cid, pl.ds(r0, rows_per_sub)])

        @pl.when(sid == NS - 1)
        def _write_tail():
            pltpu.sync_copy(acc.at[pl.ds(tail_start, tail_rows)],
                            out_hbm.at[cid, pl.ds(tail_start, tail_rows)])

    return spmm


def _tc1(x, w1):
    def body(x_ref, w_ref, o0_ref, o1_ref):
        xb = x_ref[...]
        o0_ref[...] = jnp.dot(xb, w_ref[:, :128],
                              preferred_element_type=jnp.float32,
                              precision=lax.Precision.HIGHEST)
        o1_ref[...] = jnp.dot(xb, w_ref[:, 128:],
                              preferred_element_type=jnp.float32,
                              precision=lax.Precision.HIGHEST)

    return pl.pallas_call(
        body,
        grid=(N_NODES // ROW_BLK,),
        in_specs=[
            pl.BlockSpec((ROW_BLK, NFEAT), lambda i: (i, 0)),
            pl.BlockSpec((NFEAT, NHID), lambda i: (0, 0)),
        ],
        out_specs=[
            pl.BlockSpec((ROW_BLK, 128), lambda i: (i, 0)),
            pl.BlockSpec((ROW_BLK, 128), lambda i: (i, 0)),
        ],
        out_shape=[
            jax.ShapeDtypeStruct((N_NODES, 128), jnp.float32),
            jax.ShapeDtypeStruct((N_NODES, 128), jnp.float32),
        ],
    )(x, w1)


def _tc2(pa, pb, b1, w2):
    def body(pa_ref, pb_ref, b1_ref, w2_ref, o_ref):
        ha = jnp.maximum(pa_ref[0] + pa_ref[1] + b1_ref[:, :128], 0.0)
        hb = jnp.maximum(pb_ref[0] + pb_ref[1] + b1_ref[:, 128:], 0.0)
        o_ref[...] = (
            jnp.dot(ha, w2_ref[:128, :], preferred_element_type=jnp.float32,
                    precision=lax.Precision.HIGHEST)
            + jnp.dot(hb, w2_ref[128:, :], preferred_element_type=jnp.float32,
                      precision=lax.Precision.HIGHEST))

    w2p = jnp.pad(w2, ((0, 0), (0, 128 - NCLASS)))
    return pl.pallas_call(
        body,
        grid=(N_NODES // ROW_BLK,),
        in_specs=[
            pl.BlockSpec((NC, ROW_BLK, 128), lambda i: (0, i, 0)),
            pl.BlockSpec((NC, ROW_BLK, 128), lambda i: (0, i, 0)),
            pl.BlockSpec((1, NHID), lambda i: (0, 0)),
            pl.BlockSpec((NHID, 128), lambda i: (0, 0)),
        ],
        out_specs=pl.BlockSpec((ROW_BLK, 128), lambda i: (i, 0)),
        out_shape=jax.ShapeDtypeStruct((N_NODES, 128), jnp.float32),
    )(pa, pb, b1.reshape(1, NHID), w2p)


def _tc3(p2, b2):
    def body(p_ref, b_ref, o_ref):
        o = p_ref[0, :, :NCLASS] + p_ref[1, :, :NCLASS] + b_ref[...]
        m = jnp.max(o, axis=1, keepdims=True)
        e = jnp.exp(o - m)
        o_ref[...] = e / jnp.sum(e, axis=1, keepdims=True)

    return pl.pallas_call(
        body,
        grid=(N_NODES // ROW_BLK,),
        in_specs=[
            pl.BlockSpec((NC, ROW_BLK, 128), lambda i: (0, i, 0)),
            pl.BlockSpec((1, NCLASS), lambda i: (0, 0)),
        ],
        out_specs=pl.BlockSpec((ROW_BLK, NCLASS), lambda i: (i, 0)),
        out_shape=jax.ShapeDtypeStruct((N_NODES, NCLASS), jnp.float32),
    )(p2, b2.reshape(1, NCLASS))


N0_CHUNKS = 96
N1_CHUNKS = 220


def _split_per_core(a, pad_val, nmax, q0, q1):
    e = a.shape[0]
    total = NS * (q0 + q1)
    a = jnp.pad(a, (0, total - e), constant_values=pad_val)
    a0 = a[:NS * q0].reshape(NS, q0)
    a1 = a[NS * q0:].reshape(NS, q1)
    out = jnp.zeros((NW, nmax * CHUNK), a.dtype)
    out = out.at[0::2, :q0].set(a0)
    out = out.at[1::2, :q1].set(a1)
    return out


def kernel(x, edge_index, edge_weight, W1, b1, W2, b2):
    nmax = max(N0_CHUNKS, N1_CHUNKS)
    q0 = N0_CHUNKS * CHUNK
    q1 = N1_CHUNKS * CHUNK
    src = _split_per_core(edge_index[0].astype(jnp.int32), 0, nmax, q0, q1
                          ).reshape(NW, nmax, CHUNK)
    dst = _split_per_core(edge_index[1].astype(jnp.int32), 0, nmax, q0, q1
                          ).reshape(NW, nmax, CHUNK)
    w = _split_per_core(edge_weight.astype(jnp.float32), 0.0, nmax, q0, q1)

    h0, h1 = _tc1(x, W1)
    spmm128 = _make_spmm(128, N0_CHUNKS, N1_CHUNKS)
    pa = spmm128(h0, src, dst, w)
    pb = spmm128(h1, src, dst, w)
    s2 = _tc2(pa, pb, b1, W2)
    p2 = spmm128(s2, src, dst, w)
    return _tc3(p2, b2)


# speedup vs baseline: 1.2791x; 1.2791x over previous
import functools

import jax
import jax.numpy as jnp
from jax import lax
from jax.experimental import pallas as pl
from jax.experimental.pallas import tpu as pltpu
from jax.experimental.pallas import tpu_sc as plsc

N_NODES = 10000
NFEAT = 256
NHID = 256
NCLASS = 64

NC = 2
NS = 16
NW = NC * NS
LANES = 16
CHUNK = 32

ROW_BLK = 1000


KCH = 4


def _make_spmm(d: int, n0: int, n1: int):
    nmax = max(n0, n1)
    rows_per_sub = 624
    tail_start = NS * rows_per_sub
    tail_rows = N_NODES - tail_start
    mesh = plsc.VectorSubcoreMesh(
        core_axis_name="c", subcore_axis_name="s", num_cores=NC, num_subcores=NS
    )

    @functools.partial(
        pl.kernel,
        out_type=jax.ShapeDtypeStruct((NC, N_NODES, d), jnp.float32),
        mesh=mesh,
        scratch_types=[
            pltpu.VMEM((KCH, CHUNK), jnp.int32),
            [pltpu.VMEM((CHUNK,), jnp.int32)] * KCH,
            pltpu.VMEM((KCH * CHUNK,), jnp.float32),
            [pltpu.VMEM((CHUNK, d), jnp.float32)] * KCH,
            pltpu.VMEM_SHARED((N_NODES, d), jnp.float32),
            [pltpu.SemaphoreType.DMA] * KCH,
            pltpu.SemaphoreType.DMA,
            [pltpu.SemaphoreType.DMA] * KCH,
            pltpu.SemaphoreType.DMA,
            pltpu.SemaphoreType.DMA,
        ],
    )
    def spmm(h_hbm, src_hbm, dst_hbm, w_hbm, out_hbm, src_blk,
             dst_bufs, w_all, bufs, acc, gsems, wsem, ssems, dsem, srcsem):
        cid = lax.axis_index("c")
        sid = lax.axis_index("s")
        wid = sid * NC + cid

        def _zero_row(i, carry):
            for j in range(d // LANES):
                bufs[0][i, pl.ds(j * LANES, LANES)] = jnp.zeros(
                    (LANES,), jnp.float32)
            return carry
        lax.fori_loop(0, CHUNK, _zero_row, 0)
        r0 = sid * rows_per_sub
        for t in range(19):
            pltpu.sync_copy(bufs[0],
                            acc.at[pl.ds(r0 + t * CHUNK, CHUNK)])
        pltpu.sync_copy(bufs[0].at[pl.ds(0, 16)],
                        acc.at[pl.ds(r0 + 19 * CHUNK, 16)])

        @pl.when(sid == NS - 1)
        def _zero_tail():
            pltpu.sync_copy(bufs[0].at[pl.ds(0, tail_rows)],
                            acc.at[pl.ds(tail_start, tail_rows)])

        plsc.subcore_barrier()

        def _scale(i_buf):
            for g16 in range(CHUNK // LANES):
                wreg = w_all[pl.ds(i_buf * CHUNK + g16 * LANES, LANES)]

                def _row(r, c2, _wreg=wreg, _g16=g16):
                    idx = jnp.broadcast_to(r, (LANES, 1)).astype(jnp.int32)
                    wb = lax.gather(
                        _wreg, idx,
                        lax.GatherDimensionNumbers(
                            offset_dims=(), collapsed_slice_dims=(0,),
                            start_index_map=(0,)),
                        slice_sizes=(1,),
                        mode=lax.GatherScatterMode.PROMISE_IN_BOUNDS)
                    row = _g16 * LANES + r
                    for j in range(d // LANES):
                        sl = pl.ds(j * LANES, LANES)
                        bufs[i_buf][row, sl] = bufs[i_buf][row, sl] * wb
                    return c2
                lax.fori_loop(0, LANES, _row, 0)

        def _body(t, carry):
            base = t * KCH
            sds = [pltpu.async_copy(src_hbm.at[wid, base + i],
                                    src_blk.at[i], srcsem)
                   for i in range(KCH)]
            wd = pltpu.async_copy(
                w_hbm.at[wid, pl.ds(base * CHUNK, KCH * CHUNK)], w_all, wsem)
            dds = [pltpu.async_copy(dst_hbm.at[wid, base + i],
                                    dst_bufs[i], dsem)
                   for i in range(KCH)]
            gds = []
            for i in range(KCH):
                sds[i].wait()
                gds.append(pltpu.async_copy(h_hbm.at[src_blk.at[i]], bufs[i],
                                            gsems[i]))
            wd.wait()
            for dd in dds:
                dd.wait()
            scats = []
            for i in range(KCH):
                gds[i].wait()
                _scale(i)
                if scats:
                    scats[-1].wait()
                scats.append(pltpu.async_copy(bufs[i], acc.at[dst_bufs[i]],
                                              ssems[i], add=True))
            scats[-1].wait()
            return carry
        n_my = jnp.where(cid == 0, n0 // KCH, n1 // KCH)
        lax.fori_loop(0, n_my, _body, 0)

        plsc.subcore_barrier()
        pltpu.sync_copy(acc.at[pl.ds(r0, rows_per_sub)],
                        out_hbm.at[cid, pl.ds(r0, rows_per_sub)])

        @pl.when(sid == NS - 1)
        def _write_tail():
            pltpu.sync_copy(acc.at[pl.ds(tail_start, tail_rows)],
                            out_hbm.at[cid, pl.ds(tail_start, tail_rows)])

    return spmm


def _tc1(x, w1):
    def body(x_ref, w_ref, o0_ref, o1_ref):
        xb = x_ref[...]
        o0_ref[...] = jnp.dot(xb, w_ref[:, :128],
                              preferred_element_type=jnp.float32,
                              precision=lax.Precision.HIGHEST)
        o1_ref[...] = jnp.dot(xb, w_ref[:, 128:],
                              preferred_element_type=jnp.float32,
                              precision=lax.Precision.HIGHEST)

    return pl.pallas_call(
        body,
        grid=(N_NODES // ROW_BLK,),
        in_specs=[
            pl.BlockSpec((ROW_BLK, NFEAT), lambda i: (i, 0)),
            pl.BlockSpec((NFEAT, NHID), lambda i: (0, 0)),
        ],
        out_specs=[
            pl.BlockSpec((ROW_BLK, 128), lambda i: (i, 0)),
            pl.BlockSpec((ROW_BLK, 128), lambda i: (i, 0)),
        ],
        out_shape=[
            jax.ShapeDtypeStruct((N_NODES, 128), jnp.float32),
            jax.ShapeDtypeStruct((N_NODES, 128), jnp.float32),
        ],
    )(x, w1)


def _tc2(pa, pb, b1, w2):
    def body(pa_ref, pb_ref, b1_ref, w2_ref, o_ref):
        ha = jnp.maximum(pa_ref[0] + pa_ref[1] + b1_ref[:, :128], 0.0)
        hb = jnp.maximum(pb_ref[0] + pb_ref[1] + b1_ref[:, 128:], 0.0)
        o_ref[...] = (
            jnp.dot(ha, w2_ref[:128, :], preferred_element_type=jnp.float32,
                    precision=lax.Precision.HIGHEST)
            + jnp.dot(hb, w2_ref[128:, :], preferred_element_type=jnp.float32,
                      precision=lax.Precision.HIGHEST))

    w2p = jnp.pad(w2, ((0, 0), (0, 128 - NCLASS)))
    return pl.pallas_call(
        body,
        grid=(N_NODES // ROW_BLK,),
        in_specs=[
            pl.BlockSpec((NC, ROW_BLK, 128), lambda i: (0, i, 0)),
            pl.BlockSpec((NC, ROW_BLK, 128), lambda i: (0, i, 0)),
            pl.BlockSpec((1, NHID), lambda i: (0, 0)),
            pl.BlockSpec((NHID, 128), lambda i: (0, 0)),
        ],
        out_specs=pl.BlockSpec((ROW_BLK, 128), lambda i: (i, 0)),
        out_shape=jax.ShapeDtypeStruct((N_NODES, 128), jnp.float32),
    )(pa, pb, b1.reshape(1, NHID), w2p)


def _tc3(p2, b2):
    def body(p_ref, b_ref, o_ref):
        o = p_ref[0, :, :NCLASS] + p_ref[1, :, :NCLASS] + b_ref[...]
        m = jnp.max(o, axis=1, keepdims=True)
        e = jnp.exp(o - m)
        o_ref[...] = e / jnp.sum(e, axis=1, keepdims=True)

    return pl.pallas_call(
        body,
        grid=(N_NODES // ROW_BLK,),
        in_specs=[
            pl.BlockSpec((NC, ROW_BLK, 128), lambda i: (0, i, 0)),
            pl.BlockSpec((1, NCLASS), lambda i: (0, 0)),
        ],
        out_specs=pl.BlockSpec((ROW_BLK, NCLASS), lambda i: (i, 0)),
        out_shape=jax.ShapeDtypeStruct((N_NODES, NCLASS), jnp.float32),
    )(p2, b2.reshape(1, NCLASS))


N0_CHUNKS = 220
N1_CHUNKS = 96


def _split_per_core(a, pad_val, nmax, q0, q1):
    e = a.shape[0]
    total = NS * (q0 + q1)
    a = jnp.pad(a, (0, total - e), constant_values=pad_val)
    a0 = a[:NS * q0].reshape(NS, q0)
    a1 = a[NS * q0:].reshape(NS, q1)
    out = jnp.zeros((NW, nmax * CHUNK), a.dtype)
    out = out.at[0::2, :q0].set(a0)
    out = out.at[1::2, :q1].set(a1)
    return out


def kernel(x, edge_index, edge_weight, W1, b1, W2, b2):
    nmax = max(N0_CHUNKS, N1_CHUNKS)
    q0 = N0_CHUNKS * CHUNK
    q1 = N1_CHUNKS * CHUNK
    src = _split_per_core(edge_index[0].astype(jnp.int32), 0, nmax, q0, q1
                          ).reshape(NW, nmax, CHUNK)
    dst = _split_per_core(edge_index[1].astype(jnp.int32), 0, nmax, q0, q1
                          ).reshape(NW, nmax, CHUNK)
    w = _split_per_core(edge_weight.astype(jnp.float32), 0.0, nmax, q0, q1)

    h0, h1 = _tc1(x, W1)
    spmm128 = _make_spmm(128, N0_CHUNKS, N1_CHUNKS)
    pa = spmm128(h0, src, dst, w)
    pb = spmm128(h1, src, dst, w)
    s2 = _tc2(pa, pb, b1, W2)
    p2 = spmm128(s2, src, dst, w)
    return _tc3(p2, b2)
